# baseline (speedup 1.0000x reference)
import functools

import jax
import jax.numpy as jnp
from jax import lax
from jax.experimental import pallas as pl
from jax.experimental.pallas import tpu as pltpu
from jax.experimental.pallas import tpu_sc as plsc

_VOCAB = 1000000
_EMB = 64
_OUT = 32
_B = 4096
_L = 200
_NTOK = _B * _L

_MB = 8192
_NBLK = -(-_VOCAB // _MB)

_NC = 2
_NS = 16
_NW = _NC * _NS
_PER_W = _NTOK // _NW
_CHUNK = 1024
_NCHUNK = _PER_W // _CHUNK


def _proj_body(embT_ref, w_ref, b_ref, out_ref):
    out_ref[...] = (
        lax.dot_general(
            embT_ref[...], w_ref[...], (((0,), (0,)), ((), ())),
            preferred_element_type=jnp.float32,
        )
        + b_ref[...]
    )


def _project_table(emb_table, fc_w, fc_b):
    w_pad = jnp.pad(fc_w.T, ((0, 0), (0, 128 - _OUT)))
    b_pad = jnp.pad(fc_b, (0, 128 - _OUT)).reshape(1, 128)
    wide = pl.pallas_call(
        _proj_body,
        grid=(_NBLK,),
        in_specs=[
            pl.BlockSpec((_EMB, _MB), lambda i: (0, i)),
            pl.BlockSpec((_EMB, 128), lambda i: (0, 0)),
            pl.BlockSpec((1, 128), lambda i: (0, 0)),
        ],
        out_specs=pl.BlockSpec((_MB, 128), lambda i: (i, 0)),
        out_shape=jax.ShapeDtypeStruct((_VOCAB, 128), jnp.float32),
    )(emb_table.T, w_pad, b_pad)
    return wide.reshape(4 * _VOCAB, _OUT)


_MESH = plsc.VectorSubcoreMesh(core_axis_name="c", subcore_axis_name="s")


@functools.partial(
    pl.kernel,
    mesh=_MESH,
    out_type=jax.ShapeDtypeStruct((_NTOK, _OUT), jnp.float32),
    scratch_types=[
        pltpu.VMEM((_CHUNK,), jnp.int32),
        pltpu.VMEM((_CHUNK, _OUT), jnp.float32),
        pltpu.SemaphoreType.DMA,
    ],
    compiler_params=pltpu.CompilerParams(use_tc_tiling_on_sc=False),
)
def _gather_rows(proj_hbm, idx_hbm, out_hbm, idx_v, rows_v, sem):
    wid = lax.axis_index("s") * _NC + lax.axis_index("c")
    base = wid * _PER_W

    def body(j, carry):
        off = pl.multiple_of(base + j * _CHUNK, 8)
        pltpu.sync_copy(idx_hbm.at[pl.ds(off, _CHUNK)], idx_v)
        pltpu.async_copy(proj_hbm.at[idx_v], rows_v, sem).wait()
        pltpu.sync_copy(rows_v, out_hbm.at[pl.ds(off, _CHUNK)])
        return carry

    lax.fori_loop(0, _NCHUNK, body, 0)


def kernel(inputs_ids, input_lens, emb_table, fc_w, fc_b):
    del input_lens
    proj = _project_table(emb_table, fc_w, fc_b)
    ids_flat = inputs_ids.reshape(_NTOK).astype(jnp.int32) * 4
    out = _gather_rows(proj, ids_flat)
    return out.reshape(_B, _L, _OUT)
